# baseline (speedup 1.0000x reference)
import jax
import jax.numpy as jnp
import numpy as np
from jax.experimental import pallas as pl
from jax.experimental.pallas import tpu as pltpu

_ROWS = 128
_COLS = 32768
_BLOCK_ROWS = 8

_KS0 = 0
_KS1 = 42
_KS2 = _KS0 ^ _KS1 ^ 0x1BD11BDA
_KS = (_KS0, _KS1, _KS2)
_ROT = ((13, 15, 26, 6), (17, 29, 16, 24))

_TINY = np.float32(1.1754943508222875e-38)


def _threefry_bits(idx):
    u32 = jnp.uint32
    x0 = jnp.full(idx.shape, u32(_KS0), dtype=u32)
    x1 = idx + u32(_KS1)
    for j in range(1, 6):
        for r in _ROT[(j - 1) % 2]:
            x0 = x0 + x1
            x1 = (x1 << u32(r)) | (x1 >> u32(32 - r))
            x1 = x0 ^ x1
        x0 = x0 + u32(_KS[j % 3])
        x1 = x1 + u32((_KS[(j + 1) % 3] + j) & 0xFFFFFFFF)
    return x0 ^ x1


def _sample_kernel(logits_ref, actions_ref, sum_ref):
    i = pl.program_id(0)
    l = logits_ref[:, :]

    row = jax.lax.broadcasted_iota(jnp.uint32, (_BLOCK_ROWS, _COLS), 0)
    col = jax.lax.broadcasted_iota(jnp.uint32, (_BLOCK_ROWS, _COLS), 1)
    base = (jnp.uint32(i) * jnp.uint32(_BLOCK_ROWS) + row) * jnp.uint32(_COLS)
    bits = _threefry_bits(base + col)

    f = pltpu.bitcast((bits >> jnp.uint32(9)) | jnp.uint32(0x3F800000),
                      jnp.float32) - np.float32(1.0)
    u = jnp.maximum(_TINY, f * (np.float32(1.0) - _TINY) + _TINY)
    g = -jnp.log(-jnp.log(u))

    y = l + g
    y_max = jnp.max(y, axis=1, keepdims=True)
    col_i32 = jax.lax.broadcasted_iota(jnp.int32, (_BLOCK_ROWS, _COLS), 1)
    big = jnp.int32(2**30)
    a = jnp.min(jnp.where(y == y_max, col_i32, big), axis=1)

    l_max = jnp.max(l, axis=1, keepdims=True)
    lse = l_max[:, 0] + jnp.log(jnp.sum(jnp.exp(l - l_max), axis=1))
    l_a = jnp.sum(jnp.where(col_i32 == a[:, None], l, jnp.float32(0.0)), axis=1)
    partial = jnp.sum(l_a - lse)

    actions_ref[:, :] = a[:, None]

    @pl.when(i == 0)
    def _():
        sum_ref[:, :] = jnp.zeros((1, 1), jnp.float32)

    sum_ref[:, :] += partial.reshape(1, 1)


def kernel(logits):
    grid = _ROWS // _BLOCK_ROWS
    actions, total = pl.pallas_call(
        _sample_kernel,
        grid=(grid,),
        in_specs=[pl.BlockSpec((_BLOCK_ROWS, _COLS), lambda i: (i, 0))],
        out_specs=[
            pl.BlockSpec((_BLOCK_ROWS, 1), lambda i: (i, 0)),
            pl.BlockSpec((1, 1), lambda i: (0, 0)),
        ],
        out_shape=[
            jax.ShapeDtypeStruct((_ROWS, 1), jnp.int32),
            jax.ShapeDtypeStruct((1, 1), jnp.float32),
        ],
    )(logits)
    return actions[:, 0], total[0, 0]


# speedup vs baseline: 1.0268x; 1.0268x over previous
import jax
import jax.numpy as jnp
import numpy as np
from jax.experimental import pallas as pl
from jax.experimental.pallas import tpu as pltpu

_ROWS = 128
_COLS = 32768
_BLOCK_ROWS = 8

_KS0 = 0
_KS1 = 42
_KS2 = _KS0 ^ _KS1 ^ 0x1BD11BDA
_KS = (_KS0, _KS1, _KS2)
_ROT = ((13, 15, 26, 6), (17, 29, 16, 24))

_TINY = np.float32(1.1754943508222875e-38)


def _threefry_bits(idx):
    u32 = jnp.uint32
    x0 = jnp.full(idx.shape, u32(_KS0), dtype=u32)
    x1 = idx + u32(_KS1)
    for j in range(1, 6):
        for r in _ROT[(j - 1) % 2]:
            x0 = x0 + x1
            x1 = (x1 << u32(r)) | (x1 >> u32(32 - r))
            x1 = x0 ^ x1
        x0 = x0 + u32(_KS[j % 3])
        x1 = x1 + u32((_KS[(j + 1) % 3] + j) & 0xFFFFFFFF)
    return x0 ^ x1


def _sample_kernel(logits_ref, actions_ref, sum_ref):
    i = pl.program_id(0)
    l = logits_ref[:, :]

    row = jax.lax.broadcasted_iota(jnp.uint32, (_BLOCK_ROWS, _COLS), 0)
    col = jax.lax.broadcasted_iota(jnp.uint32, (_BLOCK_ROWS, _COLS), 1)
    base = (jnp.uint32(i) * jnp.uint32(_BLOCK_ROWS) + row) * jnp.uint32(_COLS)
    bits = _threefry_bits(base + col)

    f = pltpu.bitcast((bits >> jnp.uint32(9)) | jnp.uint32(0x3F800000),
                      jnp.float32) - np.float32(1.0)
    u = jnp.maximum(_TINY, f * (np.float32(1.0) - _TINY) + _TINY)
    w = -jnp.log(u)

    e = jnp.exp(l)
    r = e / w
    r_max = jnp.max(r, axis=1, keepdims=True)
    col_i32 = jax.lax.broadcasted_iota(jnp.int32, (_BLOCK_ROWS, _COLS), 1)
    big = jnp.int32(2**30)
    a = jnp.min(jnp.where(r == r_max, col_i32, big), axis=1)

    lse = jnp.log(jnp.sum(e, axis=1))
    l_a = jnp.sum(jnp.where(col_i32 == a[:, None], l, jnp.float32(0.0)), axis=1)
    partial = jnp.sum(l_a - lse)

    actions_ref[:, :] = a[:, None]
    sum_ref[:, :, :] = partial.reshape(1, 1, 1)


def kernel(logits):
    grid = _ROWS // _BLOCK_ROWS
    actions, partials = pl.pallas_call(
        _sample_kernel,
        grid=(grid,),
        in_specs=[pl.BlockSpec((_BLOCK_ROWS, _COLS), lambda i: (i, 0))],
        out_specs=[
            pl.BlockSpec((_BLOCK_ROWS, 1), lambda i: (i, 0)),
            pl.BlockSpec((1, 1, 1), lambda i: (i, 0, 0)),
        ],
        out_shape=[
            jax.ShapeDtypeStruct((_ROWS, 1), jnp.int32),
            jax.ShapeDtypeStruct((grid, 1, 1), jnp.float32),
        ],
        compiler_params=pltpu.CompilerParams(
            dimension_semantics=("parallel",),
        ),
    )(logits)
    return actions[:, 0], jnp.sum(partials)


# speedup vs baseline: 1.1046x; 1.0757x over previous
import jax
import jax.numpy as jnp
import numpy as np
from jax.experimental import pallas as pl
from jax.experimental.pallas import tpu as pltpu

_ROWS = 128
_COLS = 32768
_BLOCK_ROWS = 8
_CHUNK = 512

_KS0 = 0
_KS1 = 42
_KS2 = _KS0 ^ _KS1 ^ 0x1BD11BDA
_KS = (_KS0, _KS1, _KS2)
_ROT = ((13, 15, 26, 6), (17, 29, 16, 24))

_TINY = np.float32(1.1754943508222875e-38)


def _threefry_bits(idx):
    u32 = jnp.uint32
    x0 = jnp.full(idx.shape, u32(_KS0), dtype=u32)
    x1 = idx + u32(_KS1)
    for j in range(1, 6):
        for r in _ROT[(j - 1) % 2]:
            x0 = x0 + x1
            x1 = (x1 << u32(r)) | (x1 >> u32(32 - r))
            x1 = x0 ^ x1
        x0 = x0 + u32(_KS[j % 3])
        x1 = x1 + u32((_KS[(j + 1) % 3] + j) & 0xFFFFFFFF)
    return x0 ^ x1


def _sample_kernel(logits_ref, actions_ref, sum_ref):
    i = pl.program_id(0)
    shape = (_BLOCK_ROWS, _CHUNK)
    row_u = jax.lax.broadcasted_iota(jnp.uint32, shape, 0)
    col_u = jax.lax.broadcasted_iota(jnp.uint32, shape, 1)
    col_i = jax.lax.broadcasted_iota(jnp.int32, shape, 1)
    rowbase = (jnp.uint32(i) * jnp.uint32(_BLOCK_ROWS) + row_u) \
        * jnp.uint32(_COLS) + col_u

    def body(c, carry):
        r_acc, c_acc, l_acc, e_acc = carry
        l = logits_ref[:, pl.ds(c * _CHUNK, _CHUNK)]
        bits = _threefry_bits(rowbase + jnp.uint32(c) * jnp.uint32(_CHUNK))

        f = pltpu.bitcast((bits >> jnp.uint32(9)) | jnp.uint32(0x3F800000),
                          jnp.float32) - np.float32(1.0)
        u = jnp.maximum(_TINY, f * (np.float32(1.0) - _TINY) + _TINY)
        w = -jnp.log(u)

        e = jnp.exp(l)
        r = e / w
        upd = r > r_acc
        r_acc = jnp.where(upd, r, r_acc)
        c_acc = jnp.where(upd, c, c_acc)
        l_acc = jnp.where(upd, l, l_acc)
        return r_acc, c_acc, l_acc, e_acc + e

    init = (
        jnp.full(shape, np.float32(-1.0)),
        jnp.zeros(shape, jnp.int32),
        jnp.zeros(shape, jnp.float32),
        jnp.zeros(shape, jnp.float32),
    )
    r_acc, c_acc, l_acc, e_acc = jax.lax.fori_loop(
        0, _COLS // _CHUNK, body, init)

    r_max = jnp.max(r_acc, axis=1, keepdims=True)
    gidx = c_acc * _CHUNK + col_i
    big = jnp.int32(2**30)
    cand = jnp.where(r_acc == r_max, gidx, big)
    a = jnp.min(cand, axis=1)
    sel = cand == a[:, None]
    l_a = jnp.sum(jnp.where(sel, l_acc, jnp.float32(0.0)), axis=1)
    lse = jnp.log(jnp.sum(e_acc, axis=1))
    partial = jnp.sum(l_a - lse)

    actions_ref[:, :] = a[:, None]
    sum_ref[:, :, :] = partial.reshape(1, 1, 1)


def kernel(logits):
    grid = _ROWS // _BLOCK_ROWS
    actions, partials = pl.pallas_call(
        _sample_kernel,
        grid=(grid,),
        in_specs=[pl.BlockSpec((_BLOCK_ROWS, _COLS), lambda i: (i, 0))],
        out_specs=[
            pl.BlockSpec((_BLOCK_ROWS, 1), lambda i: (i, 0)),
            pl.BlockSpec((1, 1, 1), lambda i: (i, 0, 0)),
        ],
        out_shape=[
            jax.ShapeDtypeStruct((_ROWS, 1), jnp.int32),
            jax.ShapeDtypeStruct((grid, 1, 1), jnp.float32),
        ],
        compiler_params=pltpu.CompilerParams(
            dimension_semantics=("parallel",),
        ),
    )(logits)
    return actions[:, 0], jnp.sum(partials)


# speedup vs baseline: 1.4759x; 1.3362x over previous
import jax
import jax.numpy as jnp
import numpy as np
from jax.experimental import pallas as pl
from jax.experimental.pallas import tpu as pltpu

_ROWS = 128
_COLS = 32768
_BLOCK_ROWS = 8
_CHUNK = 2048

_KS0 = 0
_KS1 = 42
_KS2 = _KS0 ^ _KS1 ^ 0x1BD11BDA
_KS = (_KS0, _KS1, _KS2)
_ROT = ((13, 15, 26, 6), (17, 29, 16, 24))

_TINY = np.float32(1.1754943508222875e-38)


def _threefry_bits(idx):
    u32 = jnp.uint32
    x0 = jnp.full(idx.shape, u32(_KS0), dtype=u32)
    x1 = idx + u32(_KS1)
    for j in range(1, 6):
        for r in _ROT[(j - 1) % 2]:
            x0 = x0 + x1
            x1 = (x1 << u32(r)) | (x1 >> u32(32 - r))
            x1 = x0 ^ x1
        x0 = x0 + u32(_KS[j % 3])
        x1 = x1 + u32((_KS[(j + 1) % 3] + j) & 0xFFFFFFFF)
    return x0 ^ x1


def _sample_kernel(logits_ref, actions_ref, sum_ref):
    i = pl.program_id(0)
    shape = (_BLOCK_ROWS, _CHUNK)
    row_u = jax.lax.broadcasted_iota(jnp.uint32, shape, 0)
    col_u = jax.lax.broadcasted_iota(jnp.uint32, shape, 1)
    col_i = jax.lax.broadcasted_iota(jnp.int32, shape, 1)
    rowbase = (jnp.uint32(i) * jnp.uint32(_BLOCK_ROWS) + row_u) \
        * jnp.uint32(_COLS) + col_u

    def body(c, carry):
        r_acc, c_acc, l_acc, e_acc = carry
        l = logits_ref[:, pl.ds(c * _CHUNK, _CHUNK)]
        bits = _threefry_bits(rowbase + jnp.uint32(c) * jnp.uint32(_CHUNK))

        f = pltpu.bitcast((bits >> jnp.uint32(9)) | jnp.uint32(0x3F800000),
                          jnp.float32) - np.float32(1.0)
        u = jnp.maximum(_TINY, f * (np.float32(1.0) - _TINY) + _TINY)
        w = -jnp.log(u)

        e = jnp.exp(l)
        r = e / w
        upd = r > r_acc
        r_acc = jnp.where(upd, r, r_acc)
        c_acc = jnp.where(upd, c, c_acc)
        l_acc = jnp.where(upd, l, l_acc)
        return r_acc, c_acc, l_acc, e_acc + e

    init = (
        jnp.full(shape, np.float32(-1.0)),
        jnp.zeros(shape, jnp.int32),
        jnp.zeros(shape, jnp.float32),
        jnp.zeros(shape, jnp.float32),
    )
    r_acc, c_acc, l_acc, e_acc = jax.lax.fori_loop(
        0, _COLS // _CHUNK, body, init)

    r_max = jnp.max(r_acc, axis=1, keepdims=True)
    gidx = c_acc * _CHUNK + col_i
    big = jnp.int32(2**30)
    cand = jnp.where(r_acc == r_max, gidx, big)
    a = jnp.min(cand, axis=1)
    sel = cand == a[:, None]
    l_a = jnp.sum(jnp.where(sel, l_acc, jnp.float32(0.0)), axis=1)
    lse = jnp.log(jnp.sum(e_acc, axis=1))
    partial = jnp.sum(l_a - lse)

    actions_ref[:, :] = a[:, None]
    sum_ref[:, :, :] = partial.reshape(1, 1, 1)


def kernel(logits):
    grid = _ROWS // _BLOCK_ROWS
    actions, partials = pl.pallas_call(
        _sample_kernel,
        grid=(grid,),
        in_specs=[pl.BlockSpec((_BLOCK_ROWS, _COLS), lambda i: (i, 0))],
        out_specs=[
            pl.BlockSpec((_BLOCK_ROWS, 1), lambda i: (i, 0)),
            pl.BlockSpec((1, 1, 1), lambda i: (i, 0, 0)),
        ],
        out_shape=[
            jax.ShapeDtypeStruct((_ROWS, 1), jnp.int32),
            jax.ShapeDtypeStruct((grid, 1, 1), jnp.float32),
        ],
        compiler_params=pltpu.CompilerParams(
            dimension_semantics=("parallel",),
        ),
    )(logits)
    return actions[:, 0], jnp.sum(partials)
